# baseline (device time: 341798 ns/iter reference)
import jax
import jax.numpy as jnp
from jax import lax
from jax.experimental import pallas as pl
from jax.experimental.pallas import tpu as pltpu

N_DEV = 4


def kernel(x, w_mat):
    m, _ = x.shape
    _, n = w_mat.shape
    m_chunk = m // N_DEV

    x = x.astype(jnp.bfloat16)
    w = w_mat.astype(jnp.bfloat16)

    def body(x_ref, w_ref, out_ref, send_buf, comm,
             rs_send_sems, rs_recv_sems, ag_send_sems, ag_recv_sems):
        my = lax.axis_index("i")
        right = lax.rem(my + 1, N_DEV)
        left = lax.rem(my + N_DEV - 1, N_DEV)

        barrier_sem = pltpu.get_barrier_semaphore()
        for nbr in (left, right):
            pl.semaphore_signal(
                barrier_sem, inc=1,
                device_id=(nbr,), device_id_type=pl.DeviceIdType.MESH,
            )
        pl.semaphore_wait(barrier_sem, 2)

        def chunk_dot(c):
            xs = x_ref[pl.ds(c * m_chunk, m_chunk), :]
            return lax.dot_general(
                xs, w_ref[...],
                (((1,), (0,)), ((), ())),
                preferred_element_type=jnp.float32,
            )

        for s in range(N_DEV - 1):
            c_send = lax.rem(my - s + N_DEV, N_DEV)
            p = chunk_dot(c_send)
            if s > 0:
                p = p + comm[s - 1].astype(jnp.float32)
            send_buf[...] = p.astype(jnp.bfloat16)
            rdma = pltpu.make_async_remote_copy(
                src_ref=send_buf,
                dst_ref=comm.at[s],
                send_sem=rs_send_sems.at[s],
                recv_sem=rs_recv_sems.at[s],
                device_id=(right,),
                device_id_type=pl.DeviceIdType.MESH,
            )
            rdma.start()
            rdma.wait()

        q = lax.rem(my + 1, N_DEV)
        full = chunk_dot(q) + comm[N_DEV - 2].astype(jnp.float32)
        z = full * jax.nn.sigmoid(full)
        out_ref[pl.ds(q * m_chunk, m_chunk), :] = z.astype(jnp.bfloat16)

        for g in range(N_DEV - 1):
            c_send = lax.rem(my + 1 - g + N_DEV, N_DEV)
            rdma = pltpu.make_async_remote_copy(
                src_ref=out_ref.at[pl.ds(c_send * m_chunk, m_chunk), :],
                dst_ref=out_ref.at[pl.ds(c_send * m_chunk, m_chunk), :],
                send_sem=ag_send_sems.at[g],
                recv_sem=ag_recv_sems.at[g],
                device_id=(right,),
                device_id_type=pl.DeviceIdType.MESH,
            )
            rdma.start()
            rdma.wait()

    return pl.pallas_call(
        body,
        out_shape=jax.ShapeDtypeStruct((m, n), jnp.bfloat16),
        in_specs=[
            pl.BlockSpec(memory_space=pltpu.VMEM),
            pl.BlockSpec(memory_space=pltpu.VMEM),
        ],
        out_specs=pl.BlockSpec(memory_space=pltpu.VMEM),
        scratch_shapes=[
            pltpu.VMEM((m_chunk, n), jnp.bfloat16),
            pltpu.VMEM((N_DEV - 1, m_chunk, n), jnp.bfloat16),
            pltpu.SemaphoreType.DMA((N_DEV - 1,)),
            pltpu.SemaphoreType.DMA((N_DEV - 1,)),
            pltpu.SemaphoreType.DMA((N_DEV - 1,)),
            pltpu.SemaphoreType.DMA((N_DEV - 1,)),
        ],
        compiler_params=pltpu.CompilerParams(
            collective_id=0,
            vmem_limit_bytes=100 * 1024 * 1024,
        ),
    )(x, w)


# device time: 207247 ns/iter; 1.6492x vs baseline; 1.6492x over previous
import jax
import jax.numpy as jnp
from jax import lax
from jax.experimental import pallas as pl
from jax.experimental.pallas import tpu as pltpu

N_DEV = 4


def kernel(x, w_mat):
    m, _ = x.shape
    _, n = w_mat.shape
    m_chunk = m // N_DEV
    n_half = n // 2

    x = x.astype(jnp.bfloat16)
    w = w_mat.astype(jnp.bfloat16)

    def body(x_ref, w_ref, out_ref, send_buf, comm,
             rs_send_sems, rs_recv_sems, ag_send_sems, ag_recv_sems):
        my = lax.axis_index("i")
        right = lax.rem(my + 1, N_DEV)
        left = lax.rem(my + N_DEV - 1, N_DEV)

        barrier_sem = pltpu.get_barrier_semaphore()
        for nbr in (left, right):
            pl.semaphore_signal(
                barrier_sem, inc=1,
                device_id=(nbr,), device_id_type=pl.DeviceIdType.MESH,
            )
        pl.semaphore_wait(barrier_sem, 2)

        def chunk_dot(c, col0):
            xs = x_ref[pl.ds(c * m_chunk, m_chunk), :]
            ws = w_ref[:, pl.ds(col0, n_half)]
            return lax.dot_general(
                xs, ws, (((1,), (0,)), ((), ())),
                preferred_element_type=jnp.float32,
            )

        p_cw = chunk_dot(my, 0)
        p_ccw = chunk_dot(my, n_half)
        for s in range(N_DEV - 1):
            if s > 0:
                p_cw = p_cw + comm[0, s - 1].astype(jnp.float32)
                p_ccw = p_ccw + comm[1, s - 1].astype(jnp.float32)
            send_buf[0] = p_cw.astype(jnp.bfloat16)
            send_buf[1] = p_ccw.astype(jnp.bfloat16)
            rdma_cw = pltpu.make_async_remote_copy(
                src_ref=send_buf.at[0],
                dst_ref=comm.at[0, s],
                send_sem=rs_send_sems.at[0, s],
                recv_sem=rs_recv_sems.at[0, s],
                device_id=(right,),
                device_id_type=pl.DeviceIdType.MESH,
            )
            rdma_ccw = pltpu.make_async_remote_copy(
                src_ref=send_buf.at[1],
                dst_ref=comm.at[1, s],
                send_sem=rs_send_sems.at[1, s],
                recv_sem=rs_recv_sems.at[1, s],
                device_id=(left,),
                device_id_type=pl.DeviceIdType.MESH,
            )
            rdma_cw.start()
            rdma_ccw.start()
            c_cw = lax.rem(my - s - 1 + N_DEV, N_DEV)
            c_ccw = lax.rem(my + s + 1, N_DEV)
            p_cw = chunk_dot(c_cw, 0)
            p_ccw = chunk_dot(c_ccw, n_half)
            rdma_cw.wait()
            rdma_ccw.wait()

        q_cw = lax.rem(my + 1, N_DEV)
        q_ccw = lax.rem(my + N_DEV - 1, N_DEV)
        full_cw = p_cw + comm[0, N_DEV - 2].astype(jnp.float32)
        full_ccw = p_ccw + comm[1, N_DEV - 2].astype(jnp.float32)
        z_cw = full_cw * jax.nn.sigmoid(full_cw)
        z_ccw = full_ccw * jax.nn.sigmoid(full_ccw)
        out_ref[pl.ds(q_cw * m_chunk, m_chunk), pl.ds(0, n_half)] = (
            z_cw.astype(jnp.bfloat16))
        out_ref[pl.ds(q_ccw * m_chunk, m_chunk), pl.ds(n_half, n_half)] = (
            z_ccw.astype(jnp.bfloat16))

        for g in range(N_DEV - 1):
            r_cw = lax.rem(my + 1 - g + N_DEV, N_DEV)
            r_ccw = lax.rem(my - 1 + g + N_DEV, N_DEV)
            rdma_cw = pltpu.make_async_remote_copy(
                src_ref=out_ref.at[pl.ds(r_cw * m_chunk, m_chunk),
                                   pl.ds(0, n_half)],
                dst_ref=out_ref.at[pl.ds(r_cw * m_chunk, m_chunk),
                                   pl.ds(0, n_half)],
                send_sem=ag_send_sems.at[0, g],
                recv_sem=ag_recv_sems.at[0, g],
                device_id=(right,),
                device_id_type=pl.DeviceIdType.MESH,
            )
            rdma_ccw = pltpu.make_async_remote_copy(
                src_ref=out_ref.at[pl.ds(r_ccw * m_chunk, m_chunk),
                                   pl.ds(n_half, n_half)],
                dst_ref=out_ref.at[pl.ds(r_ccw * m_chunk, m_chunk),
                                   pl.ds(n_half, n_half)],
                send_sem=ag_send_sems.at[1, g],
                recv_sem=ag_recv_sems.at[1, g],
                device_id=(left,),
                device_id_type=pl.DeviceIdType.MESH,
            )
            rdma_cw.start()
            rdma_ccw.start()
            rdma_cw.wait()
            rdma_ccw.wait()

    return pl.pallas_call(
        body,
        out_shape=jax.ShapeDtypeStruct((m, n), jnp.bfloat16),
        in_specs=[
            pl.BlockSpec(memory_space=pltpu.VMEM),
            pl.BlockSpec(memory_space=pltpu.VMEM),
        ],
        out_specs=pl.BlockSpec(memory_space=pltpu.VMEM),
        scratch_shapes=[
            pltpu.VMEM((2, m_chunk, n_half), jnp.bfloat16),
            pltpu.VMEM((2, N_DEV - 1, m_chunk, n_half), jnp.bfloat16),
            pltpu.SemaphoreType.DMA((2, N_DEV - 1)),
            pltpu.SemaphoreType.DMA((2, N_DEV - 1)),
            pltpu.SemaphoreType.DMA((2, N_DEV - 1)),
            pltpu.SemaphoreType.DMA((2, N_DEV - 1)),
        ],
        compiler_params=pltpu.CompilerParams(
            collective_id=0,
            vmem_limit_bytes=100 * 1024 * 1024,
        ),
    )(x, w)


# device time: 203585 ns/iter; 1.6789x vs baseline; 1.0180x over previous
import jax
import jax.numpy as jnp
from jax import lax
from jax.experimental import pallas as pl
from jax.experimental.pallas import tpu as pltpu

N_DEV = 4


def kernel(x, w_mat):
    m, _ = x.shape
    _, n = w_mat.shape
    m_chunk = m // N_DEV
    n_half = n // 2

    x = x.astype(jnp.bfloat16)
    w = w_mat.astype(jnp.bfloat16)

    def body(x_ref, w_ref, out_ref, send_buf, comm, ag_buf,
             rs_send_sems, rs_recv_sems, ag_send_sems, ag_recv_sems,
             copy_sems):
        my = lax.axis_index("i")
        right = lax.rem(my + 1, N_DEV)
        left = lax.rem(my + N_DEV - 1, N_DEV)

        barrier_sem = pltpu.get_barrier_semaphore()
        for nbr in (left, right):
            pl.semaphore_signal(
                barrier_sem, inc=1,
                device_id=(nbr,), device_id_type=pl.DeviceIdType.MESH,
            )
        pl.semaphore_wait(barrier_sem, 2)

        def chunk_dot(c, col0):
            xs = x_ref[pl.ds(c * m_chunk, m_chunk), :]
            ws = w_ref[:, pl.ds(col0, n_half)]
            return lax.dot_general(
                xs, ws, (((1,), (0,)), ((), ())),
                preferred_element_type=jnp.float32,
            )

        p_cw = chunk_dot(my, 0)
        p_ccw = chunk_dot(my, n_half)
        for s in range(N_DEV - 1):
            if s > 0:
                p_cw = p_cw + comm[0, s - 1].astype(jnp.float32)
                p_ccw = p_ccw + comm[1, s - 1].astype(jnp.float32)
            send_buf[0] = p_cw.astype(jnp.bfloat16)
            send_buf[1] = p_ccw.astype(jnp.bfloat16)
            rdma_cw = pltpu.make_async_remote_copy(
                src_ref=send_buf.at[0],
                dst_ref=comm.at[0, s],
                send_sem=rs_send_sems.at[0, s],
                recv_sem=rs_recv_sems.at[0, s],
                device_id=(right,),
                device_id_type=pl.DeviceIdType.MESH,
            )
            rdma_ccw = pltpu.make_async_remote_copy(
                src_ref=send_buf.at[1],
                dst_ref=comm.at[1, s],
                send_sem=rs_send_sems.at[1, s],
                recv_sem=rs_recv_sems.at[1, s],
                device_id=(left,),
                device_id_type=pl.DeviceIdType.MESH,
            )
            rdma_cw.start()
            rdma_ccw.start()
            c_cw = lax.rem(my - s - 1 + N_DEV, N_DEV)
            c_ccw = lax.rem(my + s + 1, N_DEV)
            p_cw = chunk_dot(c_cw, 0)
            p_ccw = chunk_dot(c_ccw, n_half)
            rdma_cw.wait()
            rdma_ccw.wait()

        full_cw = p_cw + comm[0, N_DEV - 2].astype(jnp.float32)
        full_ccw = p_ccw + comm[1, N_DEV - 2].astype(jnp.float32)
        ag_buf[0, 0] = (full_cw * jax.nn.sigmoid(full_cw)).astype(jnp.bfloat16)
        ag_buf[1, 0] = (full_ccw * jax.nn.sigmoid(full_ccw)).astype(jnp.bfloat16)

        def out_rows(c):
            return pl.ds(c * m_chunk, m_chunk)

        col = (pl.ds(0, n_half), pl.ds(n_half, n_half))

        def copy_out(d, slot, c):
            cp = pltpu.make_async_copy(
                ag_buf.at[d, slot],
                out_ref.at[out_rows(c), col[d]],
                copy_sems.at[d, slot],
            )
            cp.start()
            return cp

        copies = [
            copy_out(0, 0, lax.rem(my + 1, N_DEV)),
            copy_out(1, 0, lax.rem(my + N_DEV - 1, N_DEV)),
        ]

        for g in range(N_DEV - 1):
            rdma_cw = pltpu.make_async_remote_copy(
                src_ref=ag_buf.at[0, g],
                dst_ref=ag_buf.at[0, g + 1],
                send_sem=ag_send_sems.at[0, g],
                recv_sem=ag_recv_sems.at[0, g],
                device_id=(right,),
                device_id_type=pl.DeviceIdType.MESH,
            )
            rdma_ccw = pltpu.make_async_remote_copy(
                src_ref=ag_buf.at[1, g],
                dst_ref=ag_buf.at[1, g + 1],
                send_sem=ag_send_sems.at[1, g],
                recv_sem=ag_recv_sems.at[1, g],
                device_id=(left,),
                device_id_type=pl.DeviceIdType.MESH,
            )
            rdma_cw.start()
            rdma_ccw.start()
            rdma_cw.wait()
            rdma_ccw.wait()
            copies.append(
                copy_out(0, g + 1, lax.rem(my + 1 - (g + 1) + N_DEV, N_DEV)))
            copies.append(
                copy_out(1, g + 1, lax.rem(my - 1 + (g + 1) + N_DEV, N_DEV)))

        for cp in copies:
            cp.wait()

    return pl.pallas_call(
        body,
        out_shape=jax.ShapeDtypeStruct((m, n), jnp.bfloat16),
        in_specs=[
            pl.BlockSpec(memory_space=pltpu.VMEM),
            pl.BlockSpec(memory_space=pltpu.VMEM),
        ],
        out_specs=pl.BlockSpec(memory_space=pl.ANY),
        scratch_shapes=[
            pltpu.VMEM((2, m_chunk, n_half), jnp.bfloat16),
            pltpu.VMEM((2, N_DEV - 1, m_chunk, n_half), jnp.bfloat16),
            pltpu.VMEM((2, N_DEV, m_chunk, n_half), jnp.bfloat16),
            pltpu.SemaphoreType.DMA((2, N_DEV - 1)),
            pltpu.SemaphoreType.DMA((2, N_DEV - 1)),
            pltpu.SemaphoreType.DMA((2, N_DEV - 1)),
            pltpu.SemaphoreType.DMA((2, N_DEV - 1)),
            pltpu.SemaphoreType.DMA((2, N_DEV)),
        ],
        compiler_params=pltpu.CompilerParams(
            collective_id=0,
            vmem_limit_bytes=100 * 1024 * 1024,
        ),
    )(x, w)


# device time: 195136 ns/iter; 1.7516x vs baseline; 1.0433x over previous
import jax
import jax.numpy as jnp
from jax import lax
from jax.experimental import pallas as pl
from jax.experimental.pallas import tpu as pltpu

N_DEV = 4


def kernel(x, w_mat):
    m, _ = x.shape
    _, n = w_mat.shape
    m_chunk = m // N_DEV
    n_half = n // 2

    w = w_mat.astype(jnp.bfloat16)

    def body(x_ref, w_ref, out_ref, send_buf, comm, z_buf,
             rs_send_sems, rs_recv_sems, ag_send_sems, ag_recv_sems,
             copy_sems):
        my = lax.axis_index("i")
        right = lax.rem(my + 1, N_DEV)
        left = lax.rem(my + N_DEV - 1, N_DEV)

        barrier_sem = pltpu.get_barrier_semaphore()
        for nbr in (left, right):
            pl.semaphore_signal(
                barrier_sem, inc=1,
                device_id=(nbr,), device_id_type=pl.DeviceIdType.MESH,
            )
        pl.semaphore_wait(barrier_sem, 2)

        def chunk_dot(c, col0):
            xs = x_ref[pl.ds(c * m_chunk, m_chunk), :].astype(jnp.bfloat16)
            ws = w_ref[:, pl.ds(col0, n_half)]
            return lax.dot_general(
                xs, ws, (((1,), (0,)), ((), ())),
                preferred_element_type=jnp.float32,
            )

        p_cw = chunk_dot(my, 0)
        p_ccw = chunk_dot(my, n_half)
        for s in range(N_DEV - 1):
            if s > 0:
                p_cw = p_cw + comm[0, s - 1].astype(jnp.float32)
                p_ccw = p_ccw + comm[1, s - 1].astype(jnp.float32)
            send_buf[0] = p_cw.astype(jnp.bfloat16)
            send_buf[1] = p_ccw.astype(jnp.bfloat16)
            rdma_cw = pltpu.make_async_remote_copy(
                src_ref=send_buf.at[0],
                dst_ref=comm.at[0, s],
                send_sem=rs_send_sems.at[0, s],
                recv_sem=rs_recv_sems.at[0, s],
                device_id=(right,),
                device_id_type=pl.DeviceIdType.MESH,
            )
            rdma_ccw = pltpu.make_async_remote_copy(
                src_ref=send_buf.at[1],
                dst_ref=comm.at[1, s],
                send_sem=rs_send_sems.at[1, s],
                recv_sem=rs_recv_sems.at[1, s],
                device_id=(left,),
                device_id_type=pl.DeviceIdType.MESH,
            )
            rdma_cw.start()
            rdma_ccw.start()
            c_cw = lax.rem(my - s - 1 + N_DEV, N_DEV)
            c_ccw = lax.rem(my + s + 1, N_DEV)
            p_cw = chunk_dot(c_cw, 0)
            p_ccw = chunk_dot(c_ccw, n_half)
            rdma_cw.wait()
            rdma_ccw.wait()

        full_cw = p_cw + comm[0, N_DEV - 2].astype(jnp.float32)
        full_ccw = p_ccw + comm[1, N_DEV - 2].astype(jnp.float32)
        z_buf[0] = (full_cw * jax.nn.sigmoid(full_cw)).astype(jnp.bfloat16)
        z_buf[1] = (full_ccw * jax.nn.sigmoid(full_ccw)).astype(jnp.bfloat16)

        def out_rows(c):
            return pl.ds(c * m_chunk, m_chunk)

        col = (pl.ds(0, n_half), pl.ds(n_half, n_half))

        own_copies = []
        for d, c in ((0, lax.rem(my + 1, N_DEV)),
                     (1, lax.rem(my + N_DEV - 1, N_DEV))):
            cp = pltpu.make_async_copy(
                z_buf.at[d], out_ref.at[out_rows(c), col[d]], copy_sems.at[d])
            cp.start()
            own_copies.append(cp)

        for g in range(N_DEV - 1):
            r_cw = lax.rem(my + 1 - g + N_DEV, N_DEV)
            r_ccw = lax.rem(my - 1 + g + N_DEV, N_DEV)
            src_cw = z_buf.at[0] if g == 0 else out_ref.at[out_rows(r_cw), col[0]]
            src_ccw = z_buf.at[1] if g == 0 else out_ref.at[out_rows(r_ccw), col[1]]
            rdma_cw = pltpu.make_async_remote_copy(
                src_ref=src_cw,
                dst_ref=out_ref.at[out_rows(r_cw), col[0]],
                send_sem=ag_send_sems.at[0, g],
                recv_sem=ag_recv_sems.at[0, g],
                device_id=(right,),
                device_id_type=pl.DeviceIdType.MESH,
            )
            rdma_ccw = pltpu.make_async_remote_copy(
                src_ref=src_ccw,
                dst_ref=out_ref.at[out_rows(r_ccw), col[1]],
                send_sem=ag_send_sems.at[1, g],
                recv_sem=ag_recv_sems.at[1, g],
                device_id=(left,),
                device_id_type=pl.DeviceIdType.MESH,
            )
            rdma_cw.start()
            rdma_ccw.start()
            rdma_cw.wait()
            rdma_ccw.wait()

        for cp in own_copies:
            cp.wait()

    return pl.pallas_call(
        body,
        out_shape=jax.ShapeDtypeStruct((m, n), jnp.bfloat16),
        in_specs=[
            pl.BlockSpec(memory_space=pltpu.VMEM),
            pl.BlockSpec(memory_space=pltpu.VMEM),
        ],
        out_specs=pl.BlockSpec(memory_space=pl.ANY),
        scratch_shapes=[
            pltpu.VMEM((2, m_chunk, n_half), jnp.bfloat16),
            pltpu.VMEM((2, N_DEV - 1, m_chunk, n_half), jnp.bfloat16),
            pltpu.VMEM((2, m_chunk, n_half), jnp.bfloat16),
            pltpu.SemaphoreType.DMA((2, N_DEV - 1)),
            pltpu.SemaphoreType.DMA((2, N_DEV - 1)),
            pltpu.SemaphoreType.DMA((2, N_DEV - 1)),
            pltpu.SemaphoreType.DMA((2, N_DEV - 1)),
            pltpu.SemaphoreType.DMA((2,)),
        ],
        compiler_params=pltpu.CompilerParams(
            collective_id=0,
            vmem_limit_bytes=100 * 1024 * 1024,
        ),
    )(x, w)


# device time: 168770 ns/iter; 2.0252x vs baseline; 1.1562x over previous
import jax
import jax.numpy as jnp
from jax import lax
from jax.experimental import pallas as pl
from jax.experimental.pallas import tpu as pltpu

N_DEV = 4
N_HOP = N_DEV - 1
SEG = 2


def kernel(x, w_mat):
    m, _ = x.shape
    _, n = w_mat.shape
    m_chunk = m // N_DEV
    m_seg = m_chunk // SEG
    n_half = n // 2

    w = w_mat.astype(jnp.bfloat16)

    def body(x_ref, w_ref, out_ref, comm, z_buf,
             rs_send_sems, rs_recv_sems, ag_send_sems, ag_recv_sems,
             copy_sems):
        my = lax.axis_index("i")
        right = lax.rem(my + 1, N_DEV)
        left = lax.rem(my + N_DEV - 1, N_DEV)
        dev = (right, left)
        col = (pl.ds(0, n_half), pl.ds(n_half, n_half))

        barrier_sem = pltpu.get_barrier_semaphore()
        for nbr in (left, right):
            pl.semaphore_signal(
                barrier_sem, inc=1,
                device_id=(nbr,), device_id_type=pl.DeviceIdType.MESH,
            )
        pl.semaphore_wait(barrier_sem, 2)

        def seg_dot(c, k, d):
            xs = x_ref[pl.ds(c * m_chunk + k * m_seg, m_seg), :]
            ws = w_ref[:, col[d]]
            return lax.dot_general(
                xs.astype(jnp.bfloat16), ws, (((1,), (0,)), ((), ())),
                preferred_element_type=jnp.float32,
            )

        def out_seg(c, k, d):
            return out_ref.at[pl.ds(c * m_chunk + k * m_seg, m_seg), col[d]]

        def rs_chunk(d, s):
            return lax.rem(my - s + N_DEV, N_DEV) if d == 0 \
                else lax.rem(my + s, N_DEV)

        sends = []
        rs_rdma = {}
        ag_rdma = {}

        def rs_send(d, s, k, src):
            r = pltpu.make_async_remote_copy(
                src_ref=src,
                dst_ref=comm.at[d, s, k],
                send_sem=rs_send_sems.at[d, s, k],
                recv_sem=rs_recv_sems.at[d, s, k],
                device_id=(dev[d],),
                device_id_type=pl.DeviceIdType.MESH,
            )
            r.start()
            rs_rdma[(d, s, k)] = r
            sends.append(r)

        for k in range(SEG):
            for d in range(2):
                comm[d, N_HOP, k] = seg_dot(my, k, d).astype(jnp.bfloat16)
                rs_send(d, 0, k, comm.at[d, N_HOP, k])

        for s in range(1, N_HOP):
            for k in range(SEG):
                for d in range(2):
                    rs_rdma[(d, s - 1, k)].wait_recv()
                    acc = seg_dot(rs_chunk(d, s), k, d) \
                        + comm[d, s - 1, k].astype(jnp.float32)
                    comm[d, s - 1, k] = acc.astype(jnp.bfloat16)
                    rs_send(d, s, k, comm.at[d, s - 1, k])

        own = (lax.rem(my + 1, N_DEV), lax.rem(my + N_DEV - 1, N_DEV))
        copies = []
        for k in range(SEG):
            for d in range(2):
                rs_rdma[(d, N_HOP - 1, k)].wait_recv()
                full = seg_dot(own[d], k, d) \
                    + comm[d, N_HOP - 1, k].astype(jnp.float32)
                z_buf[d, k] = (full * jax.nn.sigmoid(full)).astype(jnp.bfloat16)
                cp = pltpu.make_async_copy(
                    z_buf.at[d, k], out_seg(own[d], k, d), copy_sems.at[d, k])
                cp.start()
                copies.append(cp)
                r = pltpu.make_async_remote_copy(
                    src_ref=z_buf.at[d, k],
                    dst_ref=out_seg(own[d], k, d),
                    send_sem=ag_send_sems.at[d, 0, k],
                    recv_sem=ag_recv_sems.at[d, 0, k],
                    device_id=(dev[d],),
                    device_id_type=pl.DeviceIdType.MESH,
                )
                r.start()
                ag_rdma[(d, 0, k)] = r
                sends.append(r)

        def ag_chunk(d, g):
            return lax.rem(my - g + 1 + N_DEV, N_DEV) if d == 0 \
                else lax.rem(my + g - 1, N_DEV)

        for g in range(1, N_HOP):
            for k in range(SEG):
                for d in range(2):
                    ag_rdma[(d, g - 1, k)].wait_recv()
                    c = ag_chunk(d, g)
                    r = pltpu.make_async_remote_copy(
                        src_ref=out_seg(c, k, d),
                        dst_ref=out_seg(c, k, d),
                        send_sem=ag_send_sems.at[d, g, k],
                        recv_sem=ag_recv_sems.at[d, g, k],
                        device_id=(dev[d],),
                        device_id_type=pl.DeviceIdType.MESH,
                    )
                    r.start()
                    ag_rdma[(d, g, k)] = r
                    sends.append(r)

        for k in range(SEG):
            for d in range(2):
                ag_rdma[(d, N_HOP - 1, k)].wait_recv()
        for cp in copies:
            cp.wait()
        for r in sends:
            r.wait_send()

    return pl.pallas_call(
        body,
        out_shape=jax.ShapeDtypeStruct((m, n), jnp.bfloat16),
        in_specs=[
            pl.BlockSpec(memory_space=pltpu.VMEM),
            pl.BlockSpec(memory_space=pltpu.VMEM),
        ],
        out_specs=pl.BlockSpec(memory_space=pl.ANY),
        scratch_shapes=[
            pltpu.VMEM((2, N_HOP + 1, SEG, m_seg, n_half), jnp.bfloat16),
            pltpu.VMEM((2, SEG, m_seg, n_half), jnp.bfloat16),
            pltpu.SemaphoreType.DMA((2, N_HOP, SEG)),
            pltpu.SemaphoreType.DMA((2, N_HOP, SEG)),
            pltpu.SemaphoreType.DMA((2, N_HOP, SEG)),
            pltpu.SemaphoreType.DMA((2, N_HOP, SEG)),
            pltpu.SemaphoreType.DMA((2, SEG)),
        ],
        compiler_params=pltpu.CompilerParams(
            collective_id=0,
            vmem_limit_bytes=100 * 1024 * 1024,
        ),
    )(x, w)
